# Initial kernel scaffold; baseline (speedup 1.0000x reference)
import jax
import jax.numpy as jnp
from jax.experimental import pallas as pl


def kernel(x, edge_index_l0, edge_index_l1, edge_index_l2, Wl0, bl0, Wr0, Wl1, bl1, Wr1, Wl2, bl2, Wr2):
    raise NotImplementedError("write your pallas kernel here")


# speedup vs baseline: 3.3646x; 3.3646x over previous
import functools

import jax
import jax.numpy as jnp
from jax import lax
from jax.experimental import pallas as pl
from jax.experimental.pallas import tpu as pltpu
from jax.experimental.pallas import tpu_sc as plsc

N0, N1, N2, N3 = 524288, 131072, 16384, 1024
D = 128
NC, NS, L = 2, 16, 16
G = 128


def _make_segsum(n_src, n_dst, E):
    R = min(8192, n_dst)
    npass = n_dst // R
    share = E // NS
    CH = min(2048, share)
    n_chunks = share // CH
    FB = CH + 256
    rpt = R // NS
    nzc = (rpt + 127) // 128

    mesh = plsc.VectorSubcoreMesh(core_axis_name="c", subcore_axis_name="s",
                                  num_cores=NC, num_subcores=NS)

    def body(x_hbm, src_hbm, dst_hbm, z2_hbm, z1_hbm, on_hbm,
             agg_out, cnt_out,
             acc_sh, cnt_sh, e_src, e_dst, f_src, f_dst,
             rows, zbuf, zvec, cbuf, idx_s, dl_s, ones_v, sem):
        c = lax.axis_index("c")
        s = lax.axis_index("s")
        ebase = s * share

        pltpu.sync_copy(z2_hbm, zbuf)
        pltpu.sync_copy(z1_hbm, zvec)
        pltpu.sync_copy(on_hbm, ones_v)

        def process_group():
            pltpu.async_copy(x_hbm.at[idx_s], rows, sem).wait()
            pltpu.sync_copy(rows, acc_sh.at[dl_s], add=True)
            pltpu.sync_copy(ones_v, cnt_sh.at[dl_s], add=True)

        def cbody(ci, w, lo):
            eoff = ebase + ci * CH
            pltpu.sync_copy(src_hbm.at[pl.ds(eoff, CH)], e_src)
            pltpu.sync_copy(dst_hbm.at[pl.ds(eoff, CH)], e_dst)

            def fstep(i, w):
                dv = e_dst[pl.ds(i * L, L)]
                sv = e_src[pl.ds(i * L, L)]
                m = (dv >= lo) & (dv < lo + R)
                mi = m.astype(jnp.int32)
                pos = w + plsc.cumsum(mi) - 1
                plsc.store_scatter(f_dst, [pos], dv - lo, mask=m)
                plsc.store_scatter(f_src, [pos], sv, mask=m)
                return w + jnp.sum(mi)

            w = lax.fori_loop(0, CH // L, fstep, w)

            def gbody(p):
                for j in range(G // L):
                    idx_s[pl.ds(j * L, L)] = f_src[pl.ds(p + j * L, L)]
                    dl_s[pl.ds(j * L, L)] = f_dst[pl.ds(p + j * L, L)]
                process_group()
                return p + G

            p = lax.while_loop(lambda p: p + G <= w, gbody, 0)
            for j in range(G // L):
                sv = f_src[pl.ds(p + j * L, L)]
                dv = f_dst[pl.ds(p + j * L, L)]
                f_src[pl.ds(j * L, L)] = sv
                f_dst[pl.ds(j * L, L)] = dv
            return w - p

        def pbody(pi, carry):
            pid = c + pi * NC
            lo = pid * R
            for zi in range(nzc):
                zr = min(128, rpt - zi * 128)
                pltpu.sync_copy(zbuf.at[pl.ds(0, zr)],
                                acc_sh.at[pl.ds(s * rpt + zi * 128, zr)])
            pltpu.sync_copy(zvec.at[pl.ds(0, rpt)],
                            cnt_sh.at[pl.ds(s * rpt, rpt)])
            plsc.subcore_barrier()

            w = lax.fori_loop(0, n_chunks, functools.partial(cbody, lo=lo), 0)

            @pl.when(w > 0)
            def _():
                for j in range(G // L):
                    pos = j * L + lax.iota(jnp.int32, L)
                    sv = f_src[pl.ds(j * L, L)]
                    dv = f_dst[pl.ds(j * L, L)]
                    valid = pos < w
                    idx_s[pl.ds(j * L, L)] = jnp.where(valid, sv, pos)
                    dl_s[pl.ds(j * L, L)] = jnp.where(valid, dv,
                                                      R + (pos & (L - 1)))
                process_group()

            plsc.subcore_barrier()
            pltpu.sync_copy(acc_sh.at[pl.ds(s * rpt, rpt)],
                            agg_out.at[pl.ds(lo + s * rpt, rpt)])
            pltpu.sync_copy(cnt_sh.at[pl.ds(s * rpt, rpt)],
                            cbuf.at[pl.ds(0, rpt)])
            pltpu.sync_copy(cbuf.at[pl.ds(0, rpt)],
                            cnt_out.at[pl.ds(lo + s * rpt, rpt)])
            return carry

        npc = (npass - c + 1) // NC
        lax.fori_loop(0, npc, pbody, 0)

    kern = pl.kernel(
        body,
        out_type=(jax.ShapeDtypeStruct((n_dst, D), jnp.float32),
                  jax.ShapeDtypeStruct((n_dst,), jnp.float32)),
        mesh=mesh,
        compiler_params=pltpu.CompilerParams(needs_layout_passes=False),
        scratch_types=[
            pltpu.VMEM_SHARED((R + L, D), jnp.float32),
            pltpu.VMEM_SHARED((R + L,), jnp.float32),
            pltpu.VMEM((CH,), jnp.int32),
            pltpu.VMEM((CH,), jnp.int32),
            pltpu.VMEM((FB,), jnp.int32),
            pltpu.VMEM((FB,), jnp.int32),
            pltpu.VMEM((G, D), jnp.float32),
            pltpu.VMEM((128, D), jnp.float32),
            pltpu.VMEM((512,), jnp.float32),
            pltpu.VMEM((512,), jnp.float32),
            pltpu.VMEM((G,), jnp.int32),
            pltpu.VMEM((G,), jnp.int32),
            pltpu.VMEM((G,), jnp.float32),
            pltpu.SemaphoreType.DMA,
        ],
    )

    def run(h, src, dst):
        z2 = jnp.zeros((128, D), jnp.float32)
        z1 = jnp.zeros((512,), jnp.float32)
        on = jnp.ones((G,), jnp.float32)
        return kern(h, src, dst, z2, z1, on)

    return run


def _tc_body(relu, agg, cnt, xd, wl, bl, wr, o):
    mean = agg[...] / jnp.maximum(cnt[...], 1.0)
    out = (jnp.dot(mean, wl[...], preferred_element_type=jnp.float32)
           + bl[...]
           + jnp.dot(xd[...], wr[...], preferred_element_type=jnp.float32))
    nrm = jnp.sqrt(jnp.sum(out * out, axis=1, keepdims=True))
    out = out / jnp.maximum(nrm, 1e-12)
    if relu:
        out = jnp.maximum(out, 0.0)
    o[...] = out


def _tc_layer(agg, cnt, x_full, Wl, bl, Wr, relu, n):
    B = min(1024, n)
    return pl.pallas_call(
        functools.partial(_tc_body, relu),
        grid=(n // B,),
        in_specs=[pl.BlockSpec((B, D), lambda i: (i, 0)),
                  pl.BlockSpec((B, 1), lambda i: (i, 0)),
                  pl.BlockSpec((B, D), lambda i: (i, 0)),
                  pl.BlockSpec((D, D), lambda i: (0, 0)),
                  pl.BlockSpec((1, D), lambda i: (0, 0)),
                  pl.BlockSpec((D, D), lambda i: (0, 0))],
        out_specs=pl.BlockSpec((B, D), lambda i: (i, 0)),
        out_shape=jax.ShapeDtypeStruct((n, D), jnp.float32),
    )(agg, cnt.reshape(n, 1), x_full, Wl, bl.reshape(1, D), Wr)


_segsum_l0 = _make_segsum(N0, N1, 524288)
_segsum_l1 = _make_segsum(N1, N2, 131072)
_segsum_l2 = _make_segsum(N2, N3, 16384)


def kernel(x, edge_index_l0, edge_index_l1, edge_index_l2,
           Wl0, bl0, Wr0, Wl1, bl1, Wr1, Wl2, bl2, Wr2):
    agg0, cnt0 = _segsum_l0(x, edge_index_l0[0], edge_index_l0[1])
    h1 = _tc_layer(agg0, cnt0, x, Wl0, bl0, Wr0, True, N1)
    agg1, cnt1 = _segsum_l1(h1, edge_index_l1[0], edge_index_l1[1])
    h2 = _tc_layer(agg1, cnt1, h1, Wl1, bl1, Wr1, True, N2)
    agg2, cnt2 = _segsum_l2(h2, edge_index_l2[0], edge_index_l2[1])
    l3 = _tc_layer(agg2, cnt2, h2, Wl2, bl2, Wr2, False, N3)
    return (h1, h2, l3)


# speedup vs baseline: 4.0236x; 1.1959x over previous
import functools

import jax
import jax.numpy as jnp
from jax import lax
from jax.experimental import pallas as pl
from jax.experimental.pallas import tpu as pltpu
from jax.experimental.pallas import tpu_sc as plsc

N0, N1, N2, N3 = 524288, 131072, 16384, 1024
D = 128
NC, NS, L = 2, 16, 16
G = 128


def _make_segsum(n_src, n_dst, E, R_max=8192):
    R = min(R_max, n_dst)
    npass = n_dst // R
    share = E // NS
    CH = min(2048, share)
    n_chunks = share // CH
    FB = CH + G + 2 * L
    rpt = R // NS
    nzc = (rpt + 63) // 64

    mesh = plsc.VectorSubcoreMesh(core_axis_name="c", subcore_axis_name="s",
                                  num_cores=NC, num_subcores=NS)

    def body(x_hbm, src_hbm, dst_hbm, z2_hbm, z1_hbm, on_hbm,
             agg_out, cnt_out,
             acc_sh, cnt_sh, e_src, e_dst, f_src, f_dst,
             rows0, rows1, zbuf, zvec, cbuf,
             idx_s0, dl_s0, idx_s1, dl_s1, ones_v, gsem0, gsem1):
        c = lax.axis_index("c")
        s = lax.axis_index("s")
        ebase = s * share

        pltpu.sync_copy(z2_hbm, zbuf)
        pltpu.sync_copy(z1_hbm, zvec)
        pltpu.sync_copy(on_hbm, ones_v)

        def stage(gbase, idxr, dlr):
            for j in range(G // L):
                idxr[pl.ds(j * L, L)] = f_src[pl.ds(gbase + j * L, L)]
                dlr[pl.ds(j * L, L)] = f_dst[pl.ds(gbase + j * L, L)]

        def fire(idxr, rowsr, gsem):
            pltpu.async_copy(x_hbm.at[idxr], rowsr, gsem)

        def drain(b):
            if b == 0:
                idxr, dlr, rowsr, gsem = idx_s0, dl_s0, rows0, gsem0
            else:
                idxr, dlr, rowsr, gsem = idx_s1, dl_s1, rows1, gsem1
            pltpu.make_async_copy(x_hbm.at[idxr], rowsr, gsem).wait()
            pltpu.sync_copy(rowsr, acc_sh.at[dlr], add=True)
            pltpu.sync_copy(ones_v, cnt_sh.at[dlr], add=True)

        def cbody(ci, carry, lo):
            w, par, pend = carry
            eoff = ebase + ci * CH
            pltpu.sync_copy(src_hbm.at[pl.ds(eoff, CH)], e_src)
            pltpu.sync_copy(dst_hbm.at[pl.ds(eoff, CH)], e_dst)

            def fstep(i, w):
                dv = e_dst[pl.ds(i * L, L)]
                sv = e_src[pl.ds(i * L, L)]
                m = (dv >= lo) & (dv < lo + R)
                mi = m.astype(jnp.int32)
                pos = w + plsc.cumsum(mi) - 1
                plsc.store_scatter(f_dst, [pos], dv - lo, mask=m)
                plsc.store_scatter(f_src, [pos], sv, mask=m)
                return w + jnp.sum(mi)

            w = lax.fori_loop(0, CH // L, fstep, w)

            def wbody(st):
                p, par, pend = st

                @pl.when(par == 0)
                def _():
                    stage(p, idx_s0, dl_s0)
                    fire(idx_s0, rows0, gsem0)

                @pl.when(par == 1)
                def _():
                    stage(p, idx_s1, dl_s1)
                    fire(idx_s1, rows1, gsem1)

                @pl.when((pend == 1) & (par == 0))
                def _():
                    drain(1)

                @pl.when((pend == 1) & (par == 1))
                def _():
                    drain(0)

                return (p + G, 1 - par, 1)

            p, par, pend = lax.while_loop(lambda st: st[0] + G <= w,
                                          wbody, (0, par, pend))
            for j in range(G // L):
                sv = f_src[pl.ds(p + j * L, L)]
                dv = f_dst[pl.ds(p + j * L, L)]
                f_src[pl.ds(j * L, L)] = sv
                f_dst[pl.ds(j * L, L)] = dv
            return (w - p, par, pend)

        def pbody(pi, carry):
            pid = c + pi * NC
            lo = pid * R
            for zi in range(nzc):
                zr = min(64, rpt - zi * 64)
                pltpu.sync_copy(zbuf.at[pl.ds(0, zr)],
                                acc_sh.at[pl.ds(s * rpt + zi * 64, zr)])
            pltpu.sync_copy(zvec.at[pl.ds(0, rpt)],
                            cnt_sh.at[pl.ds(s * rpt, rpt)])
            plsc.subcore_barrier()

            w, par, pend = lax.fori_loop(0, n_chunks,
                                         functools.partial(cbody, lo=lo),
                                         (0, 0, 0))

            @pl.when((pend == 1) & (par == 0))
            def _():
                drain(1)

            @pl.when((pend == 1) & (par == 1))
            def _():
                drain(0)

            @pl.when(w > 0)
            def _():
                for j in range(G // L):
                    pos = j * L + lax.iota(jnp.int32, L)
                    sv = f_src[pl.ds(j * L, L)]
                    dv = f_dst[pl.ds(j * L, L)]
                    valid = pos < w
                    idx_s0[pl.ds(j * L, L)] = jnp.where(valid, sv, pos)
                    dl_s0[pl.ds(j * L, L)] = jnp.where(valid, dv,
                                                       R + (pos & (L - 1)))
                pltpu.async_copy(x_hbm.at[idx_s0], rows0, gsem0).wait()
                pltpu.sync_copy(rows0, acc_sh.at[dl_s0], add=True)
                pltpu.sync_copy(ones_v, cnt_sh.at[dl_s0], add=True)

            plsc.subcore_barrier()
            pltpu.sync_copy(acc_sh.at[pl.ds(s * rpt, rpt)],
                            agg_out.at[pl.ds(lo + s * rpt, rpt)])
            pltpu.sync_copy(cnt_sh.at[pl.ds(s * rpt, rpt)],
                            cbuf.at[pl.ds(0, rpt)])
            pltpu.sync_copy(cbuf.at[pl.ds(0, rpt)],
                            cnt_out.at[pl.ds(lo + s * rpt, rpt)])
            return carry

        npc = (npass - c + 1) // NC
        lax.fori_loop(0, npc, pbody, 0)

    kern = pl.kernel(
        body,
        out_type=(jax.ShapeDtypeStruct((n_dst, D), jnp.float32),
                  jax.ShapeDtypeStruct((n_dst,), jnp.float32)),
        mesh=mesh,
        compiler_params=pltpu.CompilerParams(needs_layout_passes=False),
        scratch_types=[
            pltpu.VMEM_SHARED((R + L, D), jnp.float32),
            pltpu.VMEM_SHARED((R + L,), jnp.float32),
            pltpu.VMEM((CH,), jnp.int32),
            pltpu.VMEM((CH,), jnp.int32),
            pltpu.VMEM((FB,), jnp.int32),
            pltpu.VMEM((FB,), jnp.int32),
            pltpu.VMEM((G, D), jnp.float32),
            pltpu.VMEM((G, D), jnp.float32),
            pltpu.VMEM((64, D), jnp.float32),
            pltpu.VMEM((512,), jnp.float32),
            pltpu.VMEM((512,), jnp.float32),
            pltpu.VMEM((G,), jnp.int32),
            pltpu.VMEM((G,), jnp.int32),
            pltpu.VMEM((G,), jnp.int32),
            pltpu.VMEM((G,), jnp.int32),
            pltpu.VMEM((G,), jnp.float32),
            pltpu.SemaphoreType.DMA,
            pltpu.SemaphoreType.DMA,
        ],
    )

    def run(h, src, dst):
        z2 = jnp.zeros((64, D), jnp.float32)
        z1 = jnp.zeros((512,), jnp.float32)
        on = jnp.ones((G,), jnp.float32)
        return kern(h, src, dst, z2, z1, on)

    return run


def _tc_body(relu, agg, cnt, xd, wl, bl, wr, o):
    mean = agg[...] / jnp.maximum(cnt[...], 1.0)
    out = (jnp.dot(mean, wl[...], preferred_element_type=jnp.float32)
           + bl[...]
           + jnp.dot(xd[...], wr[...], preferred_element_type=jnp.float32))
    nrm = jnp.sqrt(jnp.sum(out * out, axis=1, keepdims=True))
    out = out / jnp.maximum(nrm, 1e-12)
    if relu:
        out = jnp.maximum(out, 0.0)
    o[...] = out


def _tc_layer(agg, cnt, x_full, Wl, bl, Wr, relu, n):
    B = min(1024, n)
    return pl.pallas_call(
        functools.partial(_tc_body, relu),
        grid=(n // B,),
        in_specs=[pl.BlockSpec((B, D), lambda i: (i, 0)),
                  pl.BlockSpec((B, 1), lambda i: (i, 0)),
                  pl.BlockSpec((B, D), lambda i: (i, 0)),
                  pl.BlockSpec((D, D), lambda i: (0, 0)),
                  pl.BlockSpec((1, D), lambda i: (0, 0)),
                  pl.BlockSpec((D, D), lambda i: (0, 0))],
        out_specs=pl.BlockSpec((B, D), lambda i: (i, 0)),
        out_shape=jax.ShapeDtypeStruct((n, D), jnp.float32),
    )(agg, cnt.reshape(n, 1), x_full, Wl, bl.reshape(1, D), Wr)


_segsum_l0 = _make_segsum(N0, N1, 524288, R_max=8192)
_segsum_l1 = _make_segsum(N1, N2, 131072, R_max=8192)
_segsum_l2 = _make_segsum(N2, N3, 16384)


def kernel(x, edge_index_l0, edge_index_l1, edge_index_l2,
           Wl0, bl0, Wr0, Wl1, bl1, Wr1, Wl2, bl2, Wr2):
    agg0, cnt0 = _segsum_l0(x, edge_index_l0[0], edge_index_l0[1])
    h1 = _tc_layer(agg0, cnt0, x, Wl0, bl0, Wr0, True, N1)
    agg1, cnt1 = _segsum_l1(h1, edge_index_l1[0], edge_index_l1[1])
    h2 = _tc_layer(agg1, cnt1, h1, Wl1, bl1, Wr1, True, N2)
    agg2, cnt2 = _segsum_l2(h2, edge_index_l2[0], edge_index_l2[1])
    l3 = _tc_layer(agg2, cnt2, h2, Wl2, bl2, Wr2, False, N3)
    return (h1, h2, l3)


# speedup vs baseline: 4.5373x; 1.1277x over previous
import functools

import jax
import jax.numpy as jnp
from jax import lax
from jax.experimental import pallas as pl
from jax.experimental.pallas import tpu as pltpu
from jax.experimental.pallas import tpu_sc as plsc

N0, N1, N2, N3 = 524288, 131072, 16384, 1024
D = 128
NC, NS, L = 2, 16, 16
G = 128


def _make_segsum(n_src, n_dst, E, R_max=8192):
    R = min(R_max, n_dst)
    npass = n_dst // R
    share = E // NS
    CH = min(2048, share)
    n_chunks = share // CH
    FB = CH + G + 2 * L
    rpt = R // NS
    nzc = (rpt + 63) // 64
    UF = 4

    mesh = plsc.VectorSubcoreMesh(core_axis_name="c", subcore_axis_name="s",
                                  num_cores=NC, num_subcores=NS)

    def body(x_hbm, src_hbm, dst_hbm, z2_hbm, z1_hbm, on_hbm,
             agg_out, cnt_out,
             acc_sh, cnt_sh, e_src, e_dst, f_src, f_dst,
             rows0, rows1, zbuf, zvec, cbuf,
             idx_s0, dl_s0, idx_s1, dl_s1, ones_v, gsem0, gsem1):
        c = lax.axis_index("c")
        s = lax.axis_index("s")
        ebase = s * share

        pltpu.sync_copy(z2_hbm, zbuf)
        pltpu.sync_copy(z1_hbm, zvec)
        pltpu.sync_copy(on_hbm, ones_v)

        def stage(gbase, idxr, dlr):
            for j in range(G // L):
                idxr[pl.ds(j * L, L)] = f_src[pl.ds(gbase + j * L, L)]
                dlr[pl.ds(j * L, L)] = f_dst[pl.ds(gbase + j * L, L)]

        def fire(idxr, rowsr, gsem):
            pltpu.async_copy(x_hbm.at[idxr], rowsr, gsem)

        def drain(b):
            if b == 0:
                idxr, dlr, rowsr, gsem = idx_s0, dl_s0, rows0, gsem0
            else:
                idxr, dlr, rowsr, gsem = idx_s1, dl_s1, rows1, gsem1
            pltpu.make_async_copy(x_hbm.at[idxr], rowsr, gsem).wait()
            pltpu.sync_copy(rowsr, acc_sh.at[dlr], add=True)
            pltpu.sync_copy(ones_v, cnt_sh.at[dlr], add=True)

        def cbody(ci, carry, lo):
            w, par, pend = carry
            eoff = ebase + ci * CH
            pltpu.sync_copy(src_hbm.at[pl.ds(eoff, CH)], e_src)
            pltpu.sync_copy(dst_hbm.at[pl.ds(eoff, CH)], e_dst)

            def fstep(i, w):
                base = i * (UF * L)
                us, svs, ms, css = [], [], [], []
                for k in range(UF):
                    dv = e_dst[pl.ds(base + k * L, L)]
                    sv = e_src[pl.ds(base + k * L, L)]
                    u = dv - lo
                    m = plsc.bitcast(u, jnp.uint32) < jnp.uint32(R)
                    us.append(u)
                    svs.append(sv)
                    ms.append(m)
                    css.append(plsc.cumsum(m.astype(jnp.int32)))
                for k in range(UF):
                    pos = w + css[k] - 1
                    plsc.store_scatter(f_dst, [pos], us[k], mask=ms[k])
                    plsc.store_scatter(f_src, [pos], svs[k], mask=ms[k])
                    w = w + jnp.sum(ms[k].astype(jnp.int32))
                return w

            w = lax.fori_loop(0, CH // (UF * L), fstep, w)

            def wbody(st):
                p, par, pend = st

                @pl.when(par == 0)
                def _():
                    stage(p, idx_s0, dl_s0)
                    fire(idx_s0, rows0, gsem0)

                @pl.when(par == 1)
                def _():
                    stage(p, idx_s1, dl_s1)
                    fire(idx_s1, rows1, gsem1)

                @pl.when((pend == 1) & (par == 0))
                def _():
                    drain(1)

                @pl.when((pend == 1) & (par == 1))
                def _():
                    drain(0)

                return (p + G, 1 - par, 1)

            p, par, pend = lax.while_loop(lambda st: st[0] + G <= w,
                                          wbody, (0, par, pend))
            for j in range(G // L):
                sv = f_src[pl.ds(p + j * L, L)]
                dv = f_dst[pl.ds(p + j * L, L)]
                f_src[pl.ds(j * L, L)] = sv
                f_dst[pl.ds(j * L, L)] = dv
            return (w - p, par, pend)

        def pbody(pi, carry):
            pid = c + pi * NC
            lo = pid * R
            for zi in range(nzc):
                zr = min(64, rpt - zi * 64)
                pltpu.sync_copy(zbuf.at[pl.ds(0, zr)],
                                acc_sh.at[pl.ds(s * rpt + zi * 64, zr)])
            pltpu.sync_copy(zvec.at[pl.ds(0, rpt)],
                            cnt_sh.at[pl.ds(s * rpt, rpt)])
            plsc.subcore_barrier()

            w, par, pend = lax.fori_loop(0, n_chunks,
                                         functools.partial(cbody, lo=lo),
                                         (0, 0, 0))

            @pl.when((pend == 1) & (par == 0))
            def _():
                drain(1)

            @pl.when((pend == 1) & (par == 1))
            def _():
                drain(0)

            @pl.when(w > 0)
            def _():
                for j in range(G // L):
                    pos = j * L + lax.iota(jnp.int32, L)
                    sv = f_src[pl.ds(j * L, L)]
                    dv = f_dst[pl.ds(j * L, L)]
                    valid = pos < w
                    idx_s0[pl.ds(j * L, L)] = jnp.where(valid, sv, pos)
                    dl_s0[pl.ds(j * L, L)] = jnp.where(valid, dv,
                                                       R + (pos & (L - 1)))
                pltpu.async_copy(x_hbm.at[idx_s0], rows0, gsem0).wait()
                pltpu.sync_copy(rows0, acc_sh.at[dl_s0], add=True)
                pltpu.sync_copy(ones_v, cnt_sh.at[dl_s0], add=True)

            plsc.subcore_barrier()
            pltpu.sync_copy(acc_sh.at[pl.ds(s * rpt, rpt)],
                            agg_out.at[pl.ds(lo + s * rpt, rpt)])
            pltpu.sync_copy(cnt_sh.at[pl.ds(s * rpt, rpt)],
                            cbuf.at[pl.ds(0, rpt)])
            pltpu.sync_copy(cbuf.at[pl.ds(0, rpt)],
                            cnt_out.at[pl.ds(lo + s * rpt, rpt)])
            return carry

        npc = (npass - c + 1) // NC
        lax.fori_loop(0, npc, pbody, 0)

    kern = pl.kernel(
        body,
        out_type=(jax.ShapeDtypeStruct((n_dst, D), jnp.float32),
                  jax.ShapeDtypeStruct((n_dst,), jnp.float32)),
        mesh=mesh,
        compiler_params=pltpu.CompilerParams(needs_layout_passes=False),
        scratch_types=[
            pltpu.VMEM_SHARED((R + L, D), jnp.float32),
            pltpu.VMEM_SHARED((R + L,), jnp.float32),
            pltpu.VMEM((CH,), jnp.int32),
            pltpu.VMEM((CH,), jnp.int32),
            pltpu.VMEM((FB,), jnp.int32),
            pltpu.VMEM((FB,), jnp.int32),
            pltpu.VMEM((G, D), jnp.float32),
            pltpu.VMEM((G, D), jnp.float32),
            pltpu.VMEM((64, D), jnp.float32),
            pltpu.VMEM((512,), jnp.float32),
            pltpu.VMEM((512,), jnp.float32),
            pltpu.VMEM((G,), jnp.int32),
            pltpu.VMEM((G,), jnp.int32),
            pltpu.VMEM((G,), jnp.int32),
            pltpu.VMEM((G,), jnp.int32),
            pltpu.VMEM((G,), jnp.float32),
            pltpu.SemaphoreType.DMA,
            pltpu.SemaphoreType.DMA,
        ],
    )

    def run(h, src, dst):
        z2 = jnp.zeros((64, D), jnp.float32)
        z1 = jnp.zeros((512,), jnp.float32)
        on = jnp.ones((G,), jnp.float32)
        return kern(h, src, dst, z2, z1, on)

    return run


def _tc_body(relu, agg, cnt, xd, wl, bl, wr, o):
    mean = agg[...] / jnp.maximum(cnt[...], 1.0)
    out = (jnp.dot(mean, wl[...], preferred_element_type=jnp.float32)
           + bl[...]
           + jnp.dot(xd[...], wr[...], preferred_element_type=jnp.float32))
    nrm = jnp.sqrt(jnp.sum(out * out, axis=1, keepdims=True))
    out = out / jnp.maximum(nrm, 1e-12)
    if relu:
        out = jnp.maximum(out, 0.0)
    o[...] = out


def _tc_layer(agg, cnt, x_full, Wl, bl, Wr, relu, n):
    B = min(1024, n)
    return pl.pallas_call(
        functools.partial(_tc_body, relu),
        grid=(n // B,),
        in_specs=[pl.BlockSpec((B, D), lambda i: (i, 0)),
                  pl.BlockSpec((B, 1), lambda i: (i, 0)),
                  pl.BlockSpec((B, D), lambda i: (i, 0)),
                  pl.BlockSpec((D, D), lambda i: (0, 0)),
                  pl.BlockSpec((1, D), lambda i: (0, 0)),
                  pl.BlockSpec((D, D), lambda i: (0, 0))],
        out_specs=pl.BlockSpec((B, D), lambda i: (i, 0)),
        out_shape=jax.ShapeDtypeStruct((n, D), jnp.float32),
    )(agg, cnt.reshape(n, 1), x_full, Wl, bl.reshape(1, D), Wr)


_segsum_l0 = _make_segsum(N0, N1, 524288, R_max=8192)
_segsum_l1 = _make_segsum(N1, N2, 131072, R_max=8192)
_segsum_l2 = _make_segsum(N2, N3, 16384)


def kernel(x, edge_index_l0, edge_index_l1, edge_index_l2,
           Wl0, bl0, Wr0, Wl1, bl1, Wr1, Wl2, bl2, Wr2):
    agg0, cnt0 = _segsum_l0(x, edge_index_l0[0], edge_index_l0[1])
    h1 = _tc_layer(agg0, cnt0, x, Wl0, bl0, Wr0, True, N1)
    agg1, cnt1 = _segsum_l1(h1, edge_index_l1[0], edge_index_l1[1])
    h2 = _tc_layer(agg1, cnt1, h1, Wl1, bl1, Wr1, True, N2)
    agg2, cnt2 = _segsum_l2(h2, edge_index_l2[0], edge_index_l2[1])
    l3 = _tc_layer(agg2, cnt2, h2, Wl2, bl2, Wr2, False, N3)
    return (h1, h2, l3)


# speedup vs baseline: 6.0851x; 1.3411x over previous
import functools

import jax
import jax.numpy as jnp
from jax import lax
from jax.experimental import pallas as pl
from jax.experimental.pallas import tpu as pltpu
from jax.experimental.pallas import tpu_sc as plsc

N0, N1, N2, N3 = 524288, 131072, 16384, 1024
D = 128
NC, NS, L = 2, 16, 16
G = 128


def _make_segsum(n_src, n_dst, E, R_max=8192):
    R = min(R_max, n_dst)
    npass = n_dst // R
    share = E // NS
    CH = min(2048, share)
    n_chunks = share // CH
    FB = CH + G + 2 * L
    rpt = R // NS
    nzc = (rpt + 63) // 64
    UF = 8

    mesh = plsc.VectorSubcoreMesh(core_axis_name="c", subcore_axis_name="s",
                                  num_cores=NC, num_subcores=NS)

    def body(x_hbm, src_hbm, dst_hbm, z2_hbm, z1_hbm, on_hbm,
             agg_out, cnt_out,
             acc_sh, cnt_sh, e_src0, e_dst0, e_src1, e_dst1, f_src, f_dst,
             rows0, rows1, zbuf, zvec, cbuf,
             idx_s0, dl_s0, idx_s1, dl_s1, ones_v,
             gsem0, gsem1, esem0, esem1):
        c = lax.axis_index("c")
        s = lax.axis_index("s")
        ebase = s * share

        pltpu.sync_copy(z2_hbm, zbuf)
        pltpu.sync_copy(z1_hbm, zvec)
        pltpu.sync_copy(on_hbm, ones_v)

        def stage(gbase, idxr, dlr):
            for j in range(G // L):
                idxr[pl.ds(j * L, L)] = f_src[pl.ds(gbase + j * L, L)]
                dlr[pl.ds(j * L, L)] = f_dst[pl.ds(gbase + j * L, L)]

        def fire(idxr, rowsr, gsem):
            pltpu.async_copy(x_hbm.at[idxr], rowsr, gsem)

        def drain(b):
            if b == 0:
                idxr, dlr, rowsr, gsem = idx_s0, dl_s0, rows0, gsem0
            else:
                idxr, dlr, rowsr, gsem = idx_s1, dl_s1, rows1, gsem1
            pltpu.make_async_copy(x_hbm.at[idxr], rowsr, gsem).wait()
            pltpu.sync_copy(rowsr, acc_sh.at[dlr], add=True)
            pltpu.sync_copy(ones_v, cnt_sh.at[dlr], add=True)

        def fire_eload(ci, es, ed, esem):
            eoff = ebase + ci * CH
            pltpu.async_copy(src_hbm.at[pl.ds(eoff, CH)], es, esem)
            pltpu.async_copy(dst_hbm.at[pl.ds(eoff, CH)], ed, esem)

        def wait_eload(es, ed, esem):
            pltpu.make_async_copy(src_hbm.at[pl.ds(0, CH)], es, esem).wait()
            pltpu.make_async_copy(dst_hbm.at[pl.ds(0, CH)], ed, esem).wait()

        def cbody(ci, carry, lo, e_src, e_dst, esem, e_srcn, e_dstn, esemn):
            w, par, pend = carry

            @pl.when(ci + 1 < n_chunks)
            def _():
                fire_eload(ci + 1, e_srcn, e_dstn, esemn)

            wait_eload(e_src, e_dst, esem)

            def fstep(i, w):
                base = i * (UF * L)
                us, svs, ms, css = [], [], [], []
                for k in range(UF):
                    dv = e_dst[pl.ds(base + k * L, L)]
                    sv = e_src[pl.ds(base + k * L, L)]
                    u = dv - lo
                    m = plsc.bitcast(u, jnp.uint32) < jnp.uint32(R)
                    us.append(u)
                    svs.append(sv)
                    ms.append(m)
                    css.append(plsc.cumsum(m.astype(jnp.int32)))
                for k in range(UF):
                    pos = w + css[k] - 1
                    plsc.store_scatter(f_dst, [pos], us[k], mask=ms[k])
                    plsc.store_scatter(f_src, [pos], svs[k], mask=ms[k])
                    w = w + jnp.sum(ms[k].astype(jnp.int32))
                return w

            w = lax.fori_loop(0, CH // (UF * L), fstep, w)

            def wbody(st):
                p, par, pend = st

                @pl.when(par == 0)
                def _():
                    stage(p, idx_s0, dl_s0)
                    fire(idx_s0, rows0, gsem0)

                @pl.when(par == 1)
                def _():
                    stage(p, idx_s1, dl_s1)
                    fire(idx_s1, rows1, gsem1)

                @pl.when((pend == 1) & (par == 0))
                def _():
                    drain(1)

                @pl.when((pend == 1) & (par == 1))
                def _():
                    drain(0)

                return (p + G, 1 - par, 1)

            p, par, pend = lax.while_loop(lambda st: st[0] + G <= w,
                                          wbody, (0, par, pend))
            for j in range(G // L):
                sv = f_src[pl.ds(p + j * L, L)]
                dv = f_dst[pl.ds(p + j * L, L)]
                f_src[pl.ds(j * L, L)] = sv
                f_dst[pl.ds(j * L, L)] = dv
            return (w - p, par, pend)

        def pbody(pi, carry):
            pid = c + pi * NC
            lo = pid * R
            fire_eload(0, e_src0, e_dst0, esem0)
            for zi in range(nzc):
                zr = min(64, rpt - zi * 64)
                pltpu.sync_copy(zbuf.at[pl.ds(0, zr)],
                                acc_sh.at[pl.ds(s * rpt + zi * 64, zr)])
            pltpu.sync_copy(zvec.at[pl.ds(0, rpt)],
                            cnt_sh.at[pl.ds(s * rpt, rpt)])
            plsc.subcore_barrier()

            if n_chunks == 1:
                w, par, pend = cbody(0, (0, 0, 0), lo,
                                     e_src0, e_dst0, esem0,
                                     e_src1, e_dst1, esem1)
            else:
                def c2body(ci2, carry):
                    carry = cbody(2 * ci2, carry, lo,
                                  e_src0, e_dst0, esem0,
                                  e_src1, e_dst1, esem1)
                    carry = cbody(2 * ci2 + 1, carry, lo,
                                  e_src1, e_dst1, esem1,
                                  e_src0, e_dst0, esem0)
                    return carry

                w, par, pend = lax.fori_loop(0, n_chunks // 2, c2body,
                                             (0, 0, 0))

            @pl.when((pend == 1) & (par == 0))
            def _():
                drain(1)

            @pl.when((pend == 1) & (par == 1))
            def _():
                drain(0)

            @pl.when(w > 0)
            def _():
                for j in range(G // L):
                    pos = j * L + lax.iota(jnp.int32, L)
                    sv = f_src[pl.ds(j * L, L)]
                    dv = f_dst[pl.ds(j * L, L)]
                    valid = pos < w
                    idx_s0[pl.ds(j * L, L)] = jnp.where(valid, sv, pos)
                    dl_s0[pl.ds(j * L, L)] = jnp.where(valid, dv,
                                                       R + (pos & (L - 1)))
                pltpu.async_copy(x_hbm.at[idx_s0], rows0, gsem0).wait()
                pltpu.sync_copy(rows0, acc_sh.at[dl_s0], add=True)
                pltpu.sync_copy(ones_v, cnt_sh.at[dl_s0], add=True)

            plsc.subcore_barrier()
            pltpu.sync_copy(acc_sh.at[pl.ds(s * rpt, rpt)],
                            agg_out.at[pl.ds(lo + s * rpt, rpt)])
            pltpu.sync_copy(cnt_sh.at[pl.ds(s * rpt, rpt)],
                            cbuf.at[pl.ds(0, rpt)])
            pltpu.sync_copy(cbuf.at[pl.ds(0, rpt)],
                            cnt_out.at[pl.ds(lo + s * rpt, rpt)])
            return carry

        npc = (npass - c + 1) // NC
        lax.fori_loop(0, npc, pbody, 0)

    kern = pl.kernel(
        body,
        out_type=(jax.ShapeDtypeStruct((n_dst, D), jnp.float32),
                  jax.ShapeDtypeStruct((n_dst,), jnp.float32)),
        mesh=mesh,
        compiler_params=pltpu.CompilerParams(needs_layout_passes=False),
        scratch_types=[
            pltpu.VMEM_SHARED((R + L, D), jnp.float32),
            pltpu.VMEM_SHARED((R + L,), jnp.float32),
            pltpu.VMEM((CH,), jnp.int32),
            pltpu.VMEM((CH,), jnp.int32),
            pltpu.VMEM((CH,), jnp.int32),
            pltpu.VMEM((CH,), jnp.int32),
            pltpu.VMEM((FB,), jnp.int32),
            pltpu.VMEM((FB,), jnp.int32),
            pltpu.VMEM((G, D), jnp.float32),
            pltpu.VMEM((G, D), jnp.float32),
            pltpu.VMEM((64, D), jnp.float32),
            pltpu.VMEM((512,), jnp.float32),
            pltpu.VMEM((512,), jnp.float32),
            pltpu.VMEM((G,), jnp.int32),
            pltpu.VMEM((G,), jnp.int32),
            pltpu.VMEM((G,), jnp.int32),
            pltpu.VMEM((G,), jnp.int32),
            pltpu.VMEM((G,), jnp.float32),
            pltpu.SemaphoreType.DMA,
            pltpu.SemaphoreType.DMA,
            pltpu.SemaphoreType.DMA,
            pltpu.SemaphoreType.DMA,
        ],
    )

    def run(h, src, dst):
        z2 = jnp.zeros((64, D), jnp.float32)
        z1 = jnp.zeros((512,), jnp.float32)
        on = jnp.ones((G,), jnp.float32)
        return kern(h, src, dst, z2, z1, on)

    return run


def _tc_body(relu, agg, cnt, xd, wl, bl, wr, o):
    mean = agg[...] / jnp.maximum(cnt[...], 1.0)
    out = (jnp.dot(mean, wl[...], preferred_element_type=jnp.float32)
           + bl[...]
           + jnp.dot(xd[...], wr[...], preferred_element_type=jnp.float32))
    nrm = jnp.sqrt(jnp.sum(out * out, axis=1, keepdims=True))
    out = out / jnp.maximum(nrm, 1e-12)
    if relu:
        out = jnp.maximum(out, 0.0)
    o[...] = out


def _tc_layer(agg, cnt, x_full, Wl, bl, Wr, relu, n):
    B = min(1024, n)
    return pl.pallas_call(
        functools.partial(_tc_body, relu),
        grid=(n // B,),
        in_specs=[pl.BlockSpec((B, D), lambda i: (i, 0)),
                  pl.BlockSpec((B, 1), lambda i: (i, 0)),
                  pl.BlockSpec((B, D), lambda i: (i, 0)),
                  pl.BlockSpec((D, D), lambda i: (0, 0)),
                  pl.BlockSpec((1, D), lambda i: (0, 0)),
                  pl.BlockSpec((D, D), lambda i: (0, 0))],
        out_specs=pl.BlockSpec((B, D), lambda i: (i, 0)),
        out_shape=jax.ShapeDtypeStruct((n, D), jnp.float32),
    )(agg, cnt.reshape(n, 1), x_full, Wl, bl.reshape(1, D), Wr)


_segsum_l0 = _make_segsum(N0, N1, 524288, R_max=8192)
_segsum_l1 = _make_segsum(N1, N2, 131072, R_max=8192)
_segsum_l2 = _make_segsum(N2, N3, 16384)


def kernel(x, edge_index_l0, edge_index_l1, edge_index_l2,
           Wl0, bl0, Wr0, Wl1, bl1, Wr1, Wl2, bl2, Wr2):
    agg0, cnt0 = _segsum_l0(x, edge_index_l0[0], edge_index_l0[1])
    h1 = _tc_layer(agg0, cnt0, x, Wl0, bl0, Wr0, True, N1)
    agg1, cnt1 = _segsum_l1(h1, edge_index_l1[0], edge_index_l1[1])
    h2 = _tc_layer(agg1, cnt1, h1, Wl1, bl1, Wr1, True, N2)
    agg2, cnt2 = _segsum_l2(h2, edge_index_l2[0], edge_index_l2[1])
    l3 = _tc_layer(agg2, cnt2, h2, Wl2, bl2, Wr2, False, N3)
    return (h1, h2, l3)


# speedup vs baseline: 6.0900x; 1.0008x over previous
import functools

import jax
import jax.numpy as jnp
from jax import lax
from jax.experimental import pallas as pl
from jax.experimental.pallas import tpu as pltpu
from jax.experimental.pallas import tpu_sc as plsc

N0, N1, N2, N3 = 524288, 131072, 16384, 1024
D = 128
NC, NS, L = 2, 16, 16
G = 128


def _make_segsum(n_src, n_dst, E, R_max=8192):
    R = min(R_max, n_dst)
    npass = n_dst // R
    share = E // NS
    CH = min(2048, share)
    n_chunks = share // CH
    FB = CH + G + 2 * L
    rpt = R // NS
    nzc = (rpt + 63) // 64
    UF = 8

    mesh = plsc.VectorSubcoreMesh(core_axis_name="c", subcore_axis_name="s",
                                  num_cores=NC, num_subcores=NS)

    def body(x_hbm, src_hbm, dst_hbm, z2_hbm, z1_hbm, on_hbm,
             agg_out, cnt_out,
             acc_sh, cnt_sh, e_src0, e_dst0, e_src1, e_dst1, f_src, f_dst,
             rows0, rows1, zbuf, zvec, cbuf,
             idx_s0, dl_s0, idx_s1, dl_s1, ones_v,
             gsem0, gsem1, esem0, esem1):
        c = lax.axis_index("c")
        s = lax.axis_index("s")
        ebase = s * share

        pltpu.sync_copy(z2_hbm, zbuf)
        pltpu.sync_copy(z1_hbm, zvec)
        pltpu.sync_copy(on_hbm, ones_v)

        def stage(gbase, idxr, dlr):
            for j in range(G // L):
                idxr[pl.ds(j * L, L)] = f_src[pl.ds(gbase + j * L, L)]
                dlr[pl.ds(j * L, L)] = f_dst[pl.ds(gbase + j * L, L)]

        def fire(idxr, rowsr, gsem):
            pltpu.async_copy(x_hbm.at[idxr], rowsr, gsem)

        def drain(b):
            if b == 0:
                idxr, dlr, rowsr, gsem = idx_s0, dl_s0, rows0, gsem0
            else:
                idxr, dlr, rowsr, gsem = idx_s1, dl_s1, rows1, gsem1
            pltpu.make_async_copy(x_hbm.at[idxr], rowsr, gsem).wait()
            pltpu.sync_copy(rowsr, acc_sh.at[dlr], add=True)
            pltpu.sync_copy(ones_v, cnt_sh.at[dlr], add=True)

        def fire_eload(ci, es, ed, esem):
            eoff = ebase + ci * CH
            pltpu.async_copy(src_hbm.at[pl.ds(eoff, CH)], es, esem)
            pltpu.async_copy(dst_hbm.at[pl.ds(eoff, CH)], ed, esem)

        def wait_eload(es, ed, esem):
            pltpu.make_async_copy(src_hbm.at[pl.ds(0, CH)], es, esem).wait()
            pltpu.make_async_copy(dst_hbm.at[pl.ds(0, CH)], ed, esem).wait()

        def cbody(ci, carry, lo, e_src, e_dst, esem, e_srcn, e_dstn, esemn):
            w, par, pend = carry

            @pl.when(ci + 1 < n_chunks)
            def _():
                fire_eload(ci + 1, e_srcn, e_dstn, esemn)

            wait_eload(e_src, e_dst, esem)

            def fstep(i, w):
                base = i * (UF * L)
                us, svs, ms, css = [], [], [], []
                for k in range(UF):
                    dv = e_dst[pl.ds(base + k * L, L)]
                    sv = e_src[pl.ds(base + k * L, L)]
                    u = dv - lo
                    m = plsc.bitcast(u, jnp.uint32) < jnp.uint32(R)
                    us.append(u)
                    svs.append(sv)
                    ms.append(m)
                    css.append(plsc.cumsum(m.astype(jnp.int32)))
                for k in range(UF):
                    pos = w + css[k] - 1
                    plsc.store_scatter(f_dst, [pos], us[k], mask=ms[k])
                    plsc.store_scatter(f_src, [pos], svs[k], mask=ms[k])
                    w = w + jnp.sum(ms[k].astype(jnp.int32))
                return w

            w = lax.fori_loop(0, CH // (UF * L), fstep, w)

            def wbody(st):
                p, par, pend = st

                @pl.when(par == 0)
                def _():
                    stage(p, idx_s0, dl_s0)
                    fire(idx_s0, rows0, gsem0)

                @pl.when(par == 1)
                def _():
                    stage(p, idx_s1, dl_s1)
                    fire(idx_s1, rows1, gsem1)

                @pl.when((pend == 1) & (par == 0))
                def _():
                    drain(1)

                @pl.when((pend == 1) & (par == 1))
                def _():
                    drain(0)

                return (p + G, 1 - par, 1)

            p, par, pend = lax.while_loop(lambda st: st[0] + G <= w,
                                          wbody, (0, par, pend))
            for j in range(G // L):
                sv = f_src[pl.ds(p + j * L, L)]
                dv = f_dst[pl.ds(p + j * L, L)]
                f_src[pl.ds(j * L, L)] = sv
                f_dst[pl.ds(j * L, L)] = dv
            return (w - p, par, pend)

        def pbody(pi, carry):
            pid = c + pi * NC
            lo = pid * R
            fire_eload(0, e_src0, e_dst0, esem0)
            for zi in range(nzc):
                zr = min(64, rpt - zi * 64)
                pltpu.sync_copy(zbuf.at[pl.ds(0, zr)],
                                acc_sh.at[pl.ds(s * rpt + zi * 64, zr)])
            pltpu.sync_copy(zvec.at[pl.ds(0, rpt)],
                            cnt_sh.at[pl.ds(s * rpt, rpt)])
            plsc.subcore_barrier()

            if n_chunks == 1:
                w, par, pend = cbody(0, (0, 0, 0), lo,
                                     e_src0, e_dst0, esem0,
                                     e_src1, e_dst1, esem1)
            else:
                def c2body(ci2, carry):
                    carry = cbody(2 * ci2, carry, lo,
                                  e_src0, e_dst0, esem0,
                                  e_src1, e_dst1, esem1)
                    carry = cbody(2 * ci2 + 1, carry, lo,
                                  e_src1, e_dst1, esem1,
                                  e_src0, e_dst0, esem0)
                    return carry

                w, par, pend = lax.fori_loop(0, n_chunks // 2, c2body,
                                             (0, 0, 0))

            @pl.when((pend == 1) & (par == 0))
            def _():
                drain(1)

            @pl.when((pend == 1) & (par == 1))
            def _():
                drain(0)

            @pl.when(w > 0)
            def _():
                for j in range(G // L):
                    pos = j * L + lax.iota(jnp.int32, L)
                    sv = f_src[pl.ds(j * L, L)]
                    dv = f_dst[pl.ds(j * L, L)]
                    valid = pos < w
                    idx_s0[pl.ds(j * L, L)] = jnp.where(valid, sv, pos)
                    dl_s0[pl.ds(j * L, L)] = jnp.where(valid, dv,
                                                       R + (pos & (L - 1)))
                pltpu.async_copy(x_hbm.at[idx_s0], rows0, gsem0).wait()
                pltpu.sync_copy(rows0, acc_sh.at[dl_s0], add=True)
                pltpu.sync_copy(ones_v, cnt_sh.at[dl_s0], add=True)

            plsc.subcore_barrier()
            pltpu.sync_copy(acc_sh.at[pl.ds(s * rpt, rpt)],
                            agg_out.at[pl.ds(lo + s * rpt, rpt)])
            pltpu.sync_copy(cnt_sh.at[pl.ds(s * rpt, rpt)],
                            cbuf.at[pl.ds(0, rpt)])
            pltpu.sync_copy(cbuf.at[pl.ds(0, rpt)],
                            cnt_out.at[pl.ds(lo + s * rpt, rpt)])
            return carry

        npc = (npass - c + 1) // NC
        lax.fori_loop(0, npc, pbody, 0)

    kern = pl.kernel(
        body,
        out_type=(jax.ShapeDtypeStruct((n_dst, D), jnp.float32),
                  jax.ShapeDtypeStruct((n_dst,), jnp.float32)),
        mesh=mesh,
        compiler_params=pltpu.CompilerParams(needs_layout_passes=False),
        scratch_types=[
            pltpu.VMEM_SHARED((R + L, D), jnp.float32),
            pltpu.VMEM_SHARED((R + L,), jnp.float32),
            pltpu.VMEM((CH,), jnp.int32),
            pltpu.VMEM((CH,), jnp.int32),
            pltpu.VMEM((CH,), jnp.int32),
            pltpu.VMEM((CH,), jnp.int32),
            pltpu.VMEM((FB,), jnp.int32),
            pltpu.VMEM((FB,), jnp.int32),
            pltpu.VMEM((G, D), jnp.float32),
            pltpu.VMEM((G, D), jnp.float32),
            pltpu.VMEM((64, D), jnp.float32),
            pltpu.VMEM((512,), jnp.float32),
            pltpu.VMEM((512,), jnp.float32),
            pltpu.VMEM((G,), jnp.int32),
            pltpu.VMEM((G,), jnp.int32),
            pltpu.VMEM((G,), jnp.int32),
            pltpu.VMEM((G,), jnp.int32),
            pltpu.VMEM((G,), jnp.float32),
            pltpu.SemaphoreType.DMA,
            pltpu.SemaphoreType.DMA,
            pltpu.SemaphoreType.DMA,
            pltpu.SemaphoreType.DMA,
        ],
    )

    def run(h, src, dst):
        z2 = jnp.zeros((64, D), jnp.float32)
        z1 = jnp.zeros((512,), jnp.float32)
        on = jnp.ones((G,), jnp.float32)
        return kern(h, src, dst, z2, z1, on)

    return run


def _tc_body(relu, agg, cnt, xd, wl, bl, wr, o):
    mean = agg[...] / jnp.maximum(cnt[...], 1.0)
    out = (jnp.dot(mean.astype(jnp.bfloat16), wl[...].astype(jnp.bfloat16),
                   preferred_element_type=jnp.float32)
           + bl[...]
           + jnp.dot(xd[...].astype(jnp.bfloat16),
                     wr[...].astype(jnp.bfloat16),
                     preferred_element_type=jnp.float32))
    nrm = jnp.sqrt(jnp.sum(out * out, axis=1, keepdims=True))
    out = out / jnp.maximum(nrm, 1e-12)
    if relu:
        out = jnp.maximum(out, 0.0)
    o[...] = out


def _tc_layer(agg, cnt, x_full, Wl, bl, Wr, relu, n):
    B = min(1024, n)
    return pl.pallas_call(
        functools.partial(_tc_body, relu),
        grid=(n // B,),
        in_specs=[pl.BlockSpec((B, D), lambda i: (i, 0)),
                  pl.BlockSpec((B, 1), lambda i: (i, 0)),
                  pl.BlockSpec((B, D), lambda i: (i, 0)),
                  pl.BlockSpec((D, D), lambda i: (0, 0)),
                  pl.BlockSpec((1, D), lambda i: (0, 0)),
                  pl.BlockSpec((D, D), lambda i: (0, 0))],
        out_specs=pl.BlockSpec((B, D), lambda i: (i, 0)),
        out_shape=jax.ShapeDtypeStruct((n, D), jnp.float32),
    )(agg, cnt.reshape(n, 1), x_full, Wl, bl.reshape(1, D), Wr)


_segsum_l0 = _make_segsum(N0, N1, 524288, R_max=8192)
_segsum_l1 = _make_segsum(N1, N2, 131072, R_max=8192)
_segsum_l2 = _make_segsum(N2, N3, 16384)


def kernel(x, edge_index_l0, edge_index_l1, edge_index_l2,
           Wl0, bl0, Wr0, Wl1, bl1, Wr1, Wl2, bl2, Wr2):
    agg0, cnt0 = _segsum_l0(x, edge_index_l0[0], edge_index_l0[1])
    h1 = _tc_layer(agg0, cnt0, x, Wl0, bl0, Wr0, True, N1)
    agg1, cnt1 = _segsum_l1(h1, edge_index_l1[0], edge_index_l1[1])
    h2 = _tc_layer(agg1, cnt1, h1, Wl1, bl1, Wr1, True, N2)
    agg2, cnt2 = _segsum_l2(h2, edge_index_l2[0], edge_index_l2[1])
    l3 = _tc_layer(agg2, cnt2, h2, Wl2, bl2, Wr2, False, N3)
    return (h1, h2, l3)


# speedup vs baseline: 6.1216x; 1.0052x over previous
import functools

import jax
import jax.numpy as jnp
from jax import lax
from jax.experimental import pallas as pl
from jax.experimental.pallas import tpu as pltpu
from jax.experimental.pallas import tpu_sc as plsc

N0, N1, N2, N3 = 524288, 131072, 16384, 1024
D = 128
NC, NS, L = 2, 16, 16
G = 128


def _make_segsum(n_src, n_dst, E, R_max=8192):
    R = min(R_max, n_dst)
    npass = n_dst // R
    share = E // NS
    CH = min(2048, share)
    n_chunks = share // CH
    FB = CH + G + 2 * L
    rpt = R // NS
    nzc = (rpt + 63) // 64
    UF = 8

    mesh = plsc.VectorSubcoreMesh(core_axis_name="c", subcore_axis_name="s",
                                  num_cores=NC, num_subcores=NS)

    def body(x_hbm, src_hbm, dst_hbm, z2_hbm, z1_hbm, on_hbm,
             agg_out, cnt_out,
             acc_sh, cnt_sh, e_src0, e_dst0, e_src1, e_dst1, f_src, f_dst,
             rows0, rows1, zbuf, zvec, cbuf,
             idx_s0, dl_s0, idx_s1, dl_s1, ones_v,
             gsem0, gsem1, esem0, esem1, zsem):
        c = lax.axis_index("c")
        s = lax.axis_index("s")
        ebase = s * share

        pltpu.sync_copy(z2_hbm, zbuf)
        pltpu.sync_copy(z1_hbm, zvec)
        pltpu.sync_copy(on_hbm, ones_v)

        def stage(gbase, idxr, dlr):
            for j in range(G // L):
                idxr[pl.ds(j * L, L)] = f_src[pl.ds(gbase + j * L, L)]
                dlr[pl.ds(j * L, L)] = f_dst[pl.ds(gbase + j * L, L)]

        def fire(idxr, rowsr, gsem):
            pltpu.async_copy(x_hbm.at[idxr], rowsr, gsem)

        def drain(b):
            if b == 0:
                idxr, dlr, rowsr, gsem = idx_s0, dl_s0, rows0, gsem0
            else:
                idxr, dlr, rowsr, gsem = idx_s1, dl_s1, rows1, gsem1
            pltpu.make_async_copy(x_hbm.at[idxr], rowsr, gsem).wait()
            pltpu.sync_copy(rowsr, acc_sh.at[dlr], add=True)
            pltpu.sync_copy(ones_v, cnt_sh.at[dlr], add=True)

        def fire_eload(ci, es, ed, esem):
            eoff = ebase + ci * CH
            pltpu.async_copy(src_hbm.at[pl.ds(eoff, CH)], es, esem)
            pltpu.async_copy(dst_hbm.at[pl.ds(eoff, CH)], ed, esem)

        def wait_eload(es, ed, esem):
            pltpu.make_async_copy(src_hbm.at[pl.ds(0, CH)], es, esem).wait()
            pltpu.make_async_copy(dst_hbm.at[pl.ds(0, CH)], ed, esem).wait()

        def cbody(ci, carry, lo, e_src, e_dst, esem, e_srcn, e_dstn, esemn):
            w, par, pend = carry

            @pl.when(ci + 1 < n_chunks)
            def _():
                fire_eload(ci + 1, e_srcn, e_dstn, esemn)

            wait_eload(e_src, e_dst, esem)

            def fstep(i, w):
                base = i * (UF * L)
                us, svs, ms, css = [], [], [], []
                for k in range(UF):
                    dv = e_dst[pl.ds(base + k * L, L)]
                    sv = e_src[pl.ds(base + k * L, L)]
                    u = dv - lo
                    m = plsc.bitcast(u, jnp.uint32) < jnp.uint32(R)
                    us.append(u)
                    svs.append(sv)
                    ms.append(m)
                    css.append(plsc.cumsum(m.astype(jnp.int32)))
                for k in range(UF):
                    pos = w + css[k] - 1
                    plsc.store_scatter(f_dst, [pos], us[k], mask=ms[k])
                    plsc.store_scatter(f_src, [pos], svs[k], mask=ms[k])
                    w = w + jnp.sum(ms[k].astype(jnp.int32))
                return w

            w = lax.fori_loop(0, CH // (UF * L), fstep, w)

            def wbody(st):
                p, par, pend = st

                @pl.when(par == 0)
                def _():
                    stage(p, idx_s0, dl_s0)
                    fire(idx_s0, rows0, gsem0)

                @pl.when(par == 1)
                def _():
                    stage(p, idx_s1, dl_s1)
                    fire(idx_s1, rows1, gsem1)

                @pl.when((pend == 1) & (par == 0))
                def _():
                    drain(1)

                @pl.when((pend == 1) & (par == 1))
                def _():
                    drain(0)

                return (p + G, 1 - par, 1)

            p, par, pend = lax.while_loop(lambda st: st[0] + G <= w,
                                          wbody, (0, par, pend))
            for j in range(G // L):
                sv = f_src[pl.ds(p + j * L, L)]
                dv = f_dst[pl.ds(p + j * L, L)]
                f_src[pl.ds(j * L, L)] = sv
                f_dst[pl.ds(j * L, L)] = dv
            return (w - p, par, pend)

        def pbody(pi, carry):
            pid = c + pi * NC
            lo = pid * R
            fire_eload(0, e_src0, e_dst0, esem0)
            for zi in range(nzc):
                zr = min(64, rpt - zi * 64)
                pltpu.async_copy(zbuf.at[pl.ds(0, zr)],
                                 acc_sh.at[pl.ds(s * rpt + zi * 64, zr)],
                                 zsem)
            pltpu.async_copy(zvec.at[pl.ds(0, rpt)],
                             cnt_sh.at[pl.ds(s * rpt, rpt)], zsem)
            for zi in range(nzc):
                zr = min(64, rpt - zi * 64)
                pltpu.make_async_copy(
                    zbuf.at[pl.ds(0, zr)],
                    acc_sh.at[pl.ds(s * rpt + zi * 64, zr)], zsem).wait()
            pltpu.make_async_copy(zvec.at[pl.ds(0, rpt)],
                                  cnt_sh.at[pl.ds(s * rpt, rpt)],
                                  zsem).wait()
            plsc.subcore_barrier()

            if n_chunks == 1:
                w, par, pend = cbody(0, (0, 0, 0), lo,
                                     e_src0, e_dst0, esem0,
                                     e_src1, e_dst1, esem1)
            else:
                def c2body(ci2, carry):
                    carry = cbody(2 * ci2, carry, lo,
                                  e_src0, e_dst0, esem0,
                                  e_src1, e_dst1, esem1)
                    carry = cbody(2 * ci2 + 1, carry, lo,
                                  e_src1, e_dst1, esem1,
                                  e_src0, e_dst0, esem0)
                    return carry

                w, par, pend = lax.fori_loop(0, n_chunks // 2, c2body,
                                             (0, 0, 0))

            @pl.when((pend == 1) & (par == 0))
            def _():
                drain(1)

            @pl.when((pend == 1) & (par == 1))
            def _():
                drain(0)

            @pl.when(w > 0)
            def _():
                for j in range(G // L):
                    pos = j * L + lax.iota(jnp.int32, L)
                    sv = f_src[pl.ds(j * L, L)]
                    dv = f_dst[pl.ds(j * L, L)]
                    valid = pos < w
                    idx_s0[pl.ds(j * L, L)] = jnp.where(valid, sv, pos)
                    dl_s0[pl.ds(j * L, L)] = jnp.where(valid, dv,
                                                       R + (pos & (L - 1)))
                pltpu.async_copy(x_hbm.at[idx_s0], rows0, gsem0).wait()
                pltpu.sync_copy(rows0, acc_sh.at[dl_s0], add=True)
                pltpu.sync_copy(ones_v, cnt_sh.at[dl_s0], add=True)

            plsc.subcore_barrier()
            pltpu.sync_copy(acc_sh.at[pl.ds(s * rpt, rpt)],
                            agg_out.at[pl.ds(lo + s * rpt, rpt)])
            pltpu.sync_copy(cnt_sh.at[pl.ds(s * rpt, rpt)],
                            cbuf.at[pl.ds(0, rpt)])
            pltpu.sync_copy(cbuf.at[pl.ds(0, rpt)],
                            cnt_out.at[pl.ds(lo + s * rpt, rpt)])
            return carry

        npc = (npass - c + 1) // NC
        lax.fori_loop(0, npc, pbody, 0)

    kern = pl.kernel(
        body,
        out_type=(jax.ShapeDtypeStruct((n_dst, D), jnp.float32),
                  jax.ShapeDtypeStruct((n_dst,), jnp.float32)),
        mesh=mesh,
        compiler_params=pltpu.CompilerParams(needs_layout_passes=False),
        scratch_types=[
            pltpu.VMEM_SHARED((R + L, D), jnp.float32),
            pltpu.VMEM_SHARED((R + L,), jnp.float32),
            pltpu.VMEM((CH,), jnp.int32),
            pltpu.VMEM((CH,), jnp.int32),
            pltpu.VMEM((CH,), jnp.int32),
            pltpu.VMEM((CH,), jnp.int32),
            pltpu.VMEM((FB,), jnp.int32),
            pltpu.VMEM((FB,), jnp.int32),
            pltpu.VMEM((G, D), jnp.float32),
            pltpu.VMEM((G, D), jnp.float32),
            pltpu.VMEM((64, D), jnp.float32),
            pltpu.VMEM((512,), jnp.float32),
            pltpu.VMEM((512,), jnp.float32),
            pltpu.VMEM((G,), jnp.int32),
            pltpu.VMEM((G,), jnp.int32),
            pltpu.VMEM((G,), jnp.int32),
            pltpu.VMEM((G,), jnp.int32),
            pltpu.VMEM((G,), jnp.float32),
            pltpu.SemaphoreType.DMA,
            pltpu.SemaphoreType.DMA,
            pltpu.SemaphoreType.DMA,
            pltpu.SemaphoreType.DMA,
            pltpu.SemaphoreType.DMA,
        ],
    )

    def run(h, src, dst):
        z2 = jnp.zeros((64, D), jnp.float32)
        z1 = jnp.zeros((512,), jnp.float32)
        on = jnp.ones((G,), jnp.float32)
        return kern(h, src, dst, z2, z1, on)

    return run


def _tc_body(relu, agg, cnt, xd, wl, bl, wr, o):
    mean = agg[...] / jnp.maximum(cnt[...], 1.0)
    out = (jnp.dot(mean, wl[...], preferred_element_type=jnp.float32)
           + bl[...]
           + jnp.dot(xd[...], wr[...], preferred_element_type=jnp.float32))
    nrm = jnp.sqrt(jnp.sum(out * out, axis=1, keepdims=True))
    out = out / jnp.maximum(nrm, 1e-12)
    if relu:
        out = jnp.maximum(out, 0.0)
    o[...] = out


def _tc_layer(agg, cnt, x_full, Wl, bl, Wr, relu, n):
    B = min(1024, n)
    return pl.pallas_call(
        functools.partial(_tc_body, relu),
        grid=(n // B,),
        in_specs=[pl.BlockSpec((B, D), lambda i: (i, 0)),
                  pl.BlockSpec((B, 1), lambda i: (i, 0)),
                  pl.BlockSpec((B, D), lambda i: (i, 0)),
                  pl.BlockSpec((D, D), lambda i: (0, 0)),
                  pl.BlockSpec((1, D), lambda i: (0, 0)),
                  pl.BlockSpec((D, D), lambda i: (0, 0))],
        out_specs=pl.BlockSpec((B, D), lambda i: (i, 0)),
        out_shape=jax.ShapeDtypeStruct((n, D), jnp.float32),
    )(agg, cnt.reshape(n, 1), x_full, Wl, bl.reshape(1, D), Wr)


_segsum_l0 = _make_segsum(N0, N1, 524288, R_max=8192)
_segsum_l1 = _make_segsum(N1, N2, 131072, R_max=8192)
_segsum_l2 = _make_segsum(N2, N3, 16384)


def kernel(x, edge_index_l0, edge_index_l1, edge_index_l2,
           Wl0, bl0, Wr0, Wl1, bl1, Wr1, Wl2, bl2, Wr2):
    agg0, cnt0 = _segsum_l0(x, edge_index_l0[0], edge_index_l0[1])
    h1 = _tc_layer(agg0, cnt0, x, Wl0, bl0, Wr0, True, N1)
    agg1, cnt1 = _segsum_l1(h1, edge_index_l1[0], edge_index_l1[1])
    h2 = _tc_layer(agg1, cnt1, h1, Wl1, bl1, Wr1, True, N2)
    agg2, cnt2 = _segsum_l2(h2, edge_index_l2[0], edge_index_l2[1])
    l3 = _tc_layer(agg2, cnt2, h2, Wl2, bl2, Wr2, False, N3)
    return (h1, h2, l3)


# speedup vs baseline: 6.2207x; 1.0162x over previous
import functools

import jax
import jax.numpy as jnp
from jax import lax
from jax.experimental import pallas as pl
from jax.experimental.pallas import tpu as pltpu
from jax.experimental.pallas import tpu_sc as plsc

N0, N1, N2, N3 = 524288, 131072, 16384, 1024
D = 128
NC, NS, L = 2, 16, 16
G = 128


def _make_segsum(n_src, n_dst, E, R_max=8192):
    R = min(R_max, n_dst)
    npass = n_dst // R
    share = E // NS
    CH = min(2048, share)
    n_chunks = share // CH
    FB = CH + G + 2 * L
    rpt = R // NS
    nzc = (rpt + 63) // 64
    UF = 16

    mesh = plsc.VectorSubcoreMesh(core_axis_name="c", subcore_axis_name="s",
                                  num_cores=NC, num_subcores=NS)

    def body(x_hbm, src_hbm, dst_hbm, z2_hbm, z1_hbm, on_hbm,
             agg_out, cnt_out,
             acc_sh, cnt_sh, e_src0, e_dst0, e_src1, e_dst1, f_src, f_dst,
             rows0, rows1, zbuf, zvec, cbuf,
             idx_s0, dl_s0, idx_s1, dl_s1, ones_v,
             gsem0, gsem1, esem0, esem1, zsem):
        c = lax.axis_index("c")
        s = lax.axis_index("s")
        ebase = s * share

        pltpu.sync_copy(z2_hbm, zbuf)
        pltpu.sync_copy(z1_hbm, zvec)
        pltpu.sync_copy(on_hbm, ones_v)

        def stage(gbase, idxr, dlr):
            for j in range(G // L):
                idxr[pl.ds(j * L, L)] = f_src[pl.ds(gbase + j * L, L)]
                dlr[pl.ds(j * L, L)] = f_dst[pl.ds(gbase + j * L, L)]

        def fire(idxr, rowsr, gsem):
            pltpu.async_copy(x_hbm.at[idxr], rowsr, gsem)

        def drain(b):
            if b == 0:
                idxr, dlr, rowsr, gsem = idx_s0, dl_s0, rows0, gsem0
            else:
                idxr, dlr, rowsr, gsem = idx_s1, dl_s1, rows1, gsem1
            pltpu.make_async_copy(x_hbm.at[idxr], rowsr, gsem).wait()
            pltpu.sync_copy(rowsr, acc_sh.at[dlr], add=True)
            pltpu.sync_copy(ones_v, cnt_sh.at[dlr], add=True)

        def fire_eload(ci, es, ed, esem):
            eoff = ebase + ci * CH
            pltpu.async_copy(src_hbm.at[pl.ds(eoff, CH)], es, esem)
            pltpu.async_copy(dst_hbm.at[pl.ds(eoff, CH)], ed, esem)

        def wait_eload(es, ed, esem):
            pltpu.make_async_copy(src_hbm.at[pl.ds(0, CH)], es, esem).wait()
            pltpu.make_async_copy(dst_hbm.at[pl.ds(0, CH)], ed, esem).wait()

        def cbody(ci, carry, lo, e_src, e_dst, esem, e_srcn, e_dstn, esemn):
            w, par, pend = carry

            @pl.when(ci + 1 < n_chunks)
            def _():
                fire_eload(ci + 1, e_srcn, e_dstn, esemn)

            wait_eload(e_src, e_dst, esem)

            def fstep(i, w):
                base = i * (UF * L)
                us, svs, ms, css = [], [], [], []
                for k in range(UF):
                    dv = e_dst[pl.ds(base + k * L, L)]
                    sv = e_src[pl.ds(base + k * L, L)]
                    u = dv - lo
                    m = plsc.bitcast(u, jnp.uint32) < jnp.uint32(R)
                    us.append(u)
                    svs.append(sv)
                    ms.append(m)
                    css.append(plsc.cumsum(m.astype(jnp.int32)))
                for k in range(UF):
                    pos = w + css[k] - 1
                    plsc.store_scatter(f_dst, [pos], us[k], mask=ms[k])
                    plsc.store_scatter(f_src, [pos], svs[k], mask=ms[k])
                    w = w + jnp.sum(ms[k].astype(jnp.int32))
                return w

            w = lax.fori_loop(0, CH // (UF * L), fstep, w)

            def wbody(st):
                p, par, pend = st

                @pl.when(par == 0)
                def _():
                    stage(p, idx_s0, dl_s0)
                    fire(idx_s0, rows0, gsem0)

                @pl.when(par == 1)
                def _():
                    stage(p, idx_s1, dl_s1)
                    fire(idx_s1, rows1, gsem1)

                @pl.when((pend == 1) & (par == 0))
                def _():
                    drain(1)

                @pl.when((pend == 1) & (par == 1))
                def _():
                    drain(0)

                return (p + G, 1 - par, 1)

            p, par, pend = lax.while_loop(lambda st: st[0] + G <= w,
                                          wbody, (0, par, pend))
            for j in range(G // L):
                sv = f_src[pl.ds(p + j * L, L)]
                dv = f_dst[pl.ds(p + j * L, L)]
                f_src[pl.ds(j * L, L)] = sv
                f_dst[pl.ds(j * L, L)] = dv
            return (w - p, par, pend)

        def pbody(pi, carry):
            pid = c + pi * NC
            lo = pid * R
            fire_eload(0, e_src0, e_dst0, esem0)
            for zi in range(nzc):
                zr = min(64, rpt - zi * 64)
                pltpu.async_copy(zbuf.at[pl.ds(0, zr)],
                                 acc_sh.at[pl.ds(s * rpt + zi * 64, zr)],
                                 zsem)
            pltpu.async_copy(zvec.at[pl.ds(0, rpt)],
                             cnt_sh.at[pl.ds(s * rpt, rpt)], zsem)
            for zi in range(nzc):
                zr = min(64, rpt - zi * 64)
                pltpu.make_async_copy(
                    zbuf.at[pl.ds(0, zr)],
                    acc_sh.at[pl.ds(s * rpt + zi * 64, zr)], zsem).wait()
            pltpu.make_async_copy(zvec.at[pl.ds(0, rpt)],
                                  cnt_sh.at[pl.ds(s * rpt, rpt)],
                                  zsem).wait()
            plsc.subcore_barrier()

            if n_chunks == 1:
                w, par, pend = cbody(0, (0, 0, 0), lo,
                                     e_src0, e_dst0, esem0,
                                     e_src1, e_dst1, esem1)
            else:
                def c2body(ci2, carry):
                    carry = cbody(2 * ci2, carry, lo,
                                  e_src0, e_dst0, esem0,
                                  e_src1, e_dst1, esem1)
                    carry = cbody(2 * ci2 + 1, carry, lo,
                                  e_src1, e_dst1, esem1,
                                  e_src0, e_dst0, esem0)
                    return carry

                w, par, pend = lax.fori_loop(0, n_chunks // 2, c2body,
                                             (0, 0, 0))

            @pl.when((pend == 1) & (par == 0))
            def _():
                drain(1)

            @pl.when((pend == 1) & (par == 1))
            def _():
                drain(0)

            @pl.when(w > 0)
            def _():
                for j in range(G // L):
                    pos = j * L + lax.iota(jnp.int32, L)
                    sv = f_src[pl.ds(j * L, L)]
                    dv = f_dst[pl.ds(j * L, L)]
                    valid = pos < w
                    idx_s0[pl.ds(j * L, L)] = jnp.where(valid, sv, pos)
                    dl_s0[pl.ds(j * L, L)] = jnp.where(valid, dv,
                                                       R + (pos & (L - 1)))
                pltpu.async_copy(x_hbm.at[idx_s0], rows0, gsem0).wait()
                pltpu.sync_copy(rows0, acc_sh.at[dl_s0], add=True)
                pltpu.sync_copy(ones_v, cnt_sh.at[dl_s0], add=True)

            plsc.subcore_barrier()
            pltpu.sync_copy(acc_sh.at[pl.ds(s * rpt, rpt)],
                            agg_out.at[pl.ds(lo + s * rpt, rpt)])
            pltpu.sync_copy(cnt_sh.at[pl.ds(s * rpt, rpt)],
                            cbuf.at[pl.ds(0, rpt)])
            pltpu.sync_copy(cbuf.at[pl.ds(0, rpt)],
                            cnt_out.at[pl.ds(lo + s * rpt, rpt)])
            return carry

        npc = (npass - c + 1) // NC
        lax.fori_loop(0, npc, pbody, 0)

    kern = pl.kernel(
        body,
        out_type=(jax.ShapeDtypeStruct((n_dst, D), jnp.float32),
                  jax.ShapeDtypeStruct((n_dst,), jnp.float32)),
        mesh=mesh,
        compiler_params=pltpu.CompilerParams(needs_layout_passes=False),
        scratch_types=[
            pltpu.VMEM_SHARED((R + L, D), jnp.float32),
            pltpu.VMEM_SHARED((R + L,), jnp.float32),
            pltpu.VMEM((CH,), jnp.int32),
            pltpu.VMEM((CH,), jnp.int32),
            pltpu.VMEM((CH,), jnp.int32),
            pltpu.VMEM((CH,), jnp.int32),
            pltpu.VMEM((FB,), jnp.int32),
            pltpu.VMEM((FB,), jnp.int32),
            pltpu.VMEM((G, D), jnp.float32),
            pltpu.VMEM((G, D), jnp.float32),
            pltpu.VMEM((64, D), jnp.float32),
            pltpu.VMEM((512,), jnp.float32),
            pltpu.VMEM((512,), jnp.float32),
            pltpu.VMEM((G,), jnp.int32),
            pltpu.VMEM((G,), jnp.int32),
            pltpu.VMEM((G,), jnp.int32),
            pltpu.VMEM((G,), jnp.int32),
            pltpu.VMEM((G,), jnp.float32),
            pltpu.SemaphoreType.DMA,
            pltpu.SemaphoreType.DMA,
            pltpu.SemaphoreType.DMA,
            pltpu.SemaphoreType.DMA,
            pltpu.SemaphoreType.DMA,
        ],
    )

    def run(h, src, dst):
        z2 = jnp.zeros((64, D), jnp.float32)
        z1 = jnp.zeros((512,), jnp.float32)
        on = jnp.ones((G,), jnp.float32)
        return kern(h, src, dst, z2, z1, on)

    return run


def _tc_body(relu, agg, cnt, xd, wl, bl, wr, o):
    mean = agg[...] / jnp.maximum(cnt[...], 1.0)
    out = (jnp.dot(mean, wl[...], preferred_element_type=jnp.float32)
           + bl[...]
           + jnp.dot(xd[...], wr[...], preferred_element_type=jnp.float32))
    nrm = jnp.sqrt(jnp.sum(out * out, axis=1, keepdims=True))
    out = out / jnp.maximum(nrm, 1e-12)
    if relu:
        out = jnp.maximum(out, 0.0)
    o[...] = out


def _tc_layer(agg, cnt, x_full, Wl, bl, Wr, relu, n):
    B = min(1024, n)
    return pl.pallas_call(
        functools.partial(_tc_body, relu),
        grid=(n // B,),
        in_specs=[pl.BlockSpec((B, D), lambda i: (i, 0)),
                  pl.BlockSpec((B, 1), lambda i: (i, 0)),
                  pl.BlockSpec((B, D), lambda i: (i, 0)),
                  pl.BlockSpec((D, D), lambda i: (0, 0)),
                  pl.BlockSpec((1, D), lambda i: (0, 0)),
                  pl.BlockSpec((D, D), lambda i: (0, 0))],
        out_specs=pl.BlockSpec((B, D), lambda i: (i, 0)),
        out_shape=jax.ShapeDtypeStruct((n, D), jnp.float32),
    )(agg, cnt.reshape(n, 1), x_full, Wl, bl.reshape(1, D), Wr)


_segsum_l0 = _make_segsum(N0, N1, 524288, R_max=8192)
_segsum_l1 = _make_segsum(N1, N2, 131072, R_max=8192)
_segsum_l2 = _make_segsum(N2, N3, 16384)


def kernel(x, edge_index_l0, edge_index_l1, edge_index_l2,
           Wl0, bl0, Wr0, Wl1, bl1, Wr1, Wl2, bl2, Wr2):
    agg0, cnt0 = _segsum_l0(x, edge_index_l0[0], edge_index_l0[1])
    h1 = _tc_layer(agg0, cnt0, x, Wl0, bl0, Wr0, True, N1)
    agg1, cnt1 = _segsum_l1(h1, edge_index_l1[0], edge_index_l1[1])
    h2 = _tc_layer(agg1, cnt1, h1, Wl1, bl1, Wr1, True, N2)
    agg2, cnt2 = _segsum_l2(h2, edge_index_l2[0], edge_index_l2[1])
    l3 = _tc_layer(agg2, cnt2, h2, Wl2, bl2, Wr2, False, N3)
    return (h1, h2, l3)
